# baseline (speedup 1.0000x reference)
import functools

import jax
import jax.numpy as jnp
from jax import lax
from jax.experimental import pallas as pl
from jax.experimental.pallas import tpu as pltpu
from jax.experimental.pallas import tpu_sc as plsc

BT = 256


@functools.lru_cache(maxsize=None)
def _make_row_gather(V, D, B):
    info = plsc.get_sparse_core_info()
    NC, NS = info.num_cores, info.num_subcores
    NW = NC * NS
    assert B % NW == 0
    b_per_w = B // NW
    C = min(64, b_per_w)
    assert b_per_w % C == 0 and (b_per_w % 8 == 0)
    n_chunks = b_per_w // C
    mesh = plsc.VectorSubcoreMesh(core_axis_name="c", subcore_axis_name="s")

    @functools.partial(
        pl.kernel,
        mesh=mesh,
        out_type=jax.ShapeDtypeStruct((B, D), jnp.float32),
        scratch_types=[
            pltpu.VMEM((C,), jnp.int32),
            pltpu.VMEM((C, D), jnp.float32),
            pltpu.SemaphoreType.DMA,
        ],
    )
    def gather(table_hbm, idx_hbm, out_hbm, idx_v, rows_v, sem):
        wid = lax.axis_index("s") * NC + lax.axis_index("c")
        base = wid * b_per_w
        for c in range(n_chunks):
            off = base + c * C
            pltpu.sync_copy(idx_hbm.at[pl.ds(off, C)], idx_v)
            pltpu.async_copy(table_hbm.at[idx_v], rows_v, sem).wait()
            pltpu.sync_copy(rows_v, out_hbm.at[pl.ds(off, C)])

    return gather


def _ffn_body(ts_ref, x_hbm, w1_ref, w3_ref, w2_ref, y_hbm,
              xa_ref, xb_ref, yt_ref, sema, semb, semy):
    e = pl.program_id(0)
    t0 = ts_ref[e]
    t1 = ts_ref[e + 1]
    dn = (((1,), (1,)), ((), ()))

    def x_copy(t, buf, sem):
        return pltpu.make_async_copy(x_hbm.at[pl.ds(t * BT, BT)], buf, sem)

    def ffn(xv):
        h1 = lax.dot_general(xv, w1_ref[0], dn,
                             preferred_element_type=jnp.float32)
        h3 = lax.dot_general(xv, w3_ref[0], dn,
                             preferred_element_type=jnp.float32)
        act = h1 * jax.nn.sigmoid(h1) * h3
        return lax.dot_general(act, w2_ref[0], dn,
                               preferred_element_type=jnp.float32)

    @pl.when(t1 > t0)
    def _():
        x_copy(t0, xa_ref, sema).start()

    def body(i, _):
        t = t0 + i * 2

        @pl.when(t + 1 < t1)
        def _():
            x_copy(t + 1, xb_ref, semb).start()

        x_copy(t, xa_ref, sema).wait()
        yt_ref[...] = ffn(xa_ref[...])
        ycp = pltpu.make_async_copy(yt_ref, y_hbm.at[pl.ds(t * BT, BT)], semy)
        ycp.start()

        @pl.when(t + 2 < t1)
        def _():
            x_copy(t + 2, xa_ref, sema).start()

        @pl.when(t + 1 < t1)
        def _():
            x_copy(t + 1, xb_ref, semb).wait()
            res = ffn(xb_ref[...])
            ycp.wait()
            yt_ref[...] = res
            cp2 = pltpu.make_async_copy(
                yt_ref, y_hbm.at[pl.ds((t + 1) * BT, BT)], semy)
            cp2.start()
            cp2.wait()

        @pl.when(t + 1 >= t1)
        def _():
            ycp.wait()

        return 0

    npairs = (t1 - t0 + 1) // 2
    lax.fori_loop(0, npairs, body, 0)


def kernel(x, expert_indices, w13, w2):
    T, D = x.shape
    A = expert_indices.shape[1]
    E = w13.shape[0]
    I = w2.shape[2]
    N = T * A
    MAX_TILES = N // BT + E
    NP = MAX_TILES * BT

    idx_flat = expert_indices.reshape(N).astype(jnp.int32)
    counts = jnp.bincount(idx_flat, length=E).astype(jnp.int32)
    tiles_per_e = (counts + BT - 1) // BT
    cum_tiles = jnp.cumsum(tiles_per_e)
    padded_start = (cum_tiles - tiles_per_e) * BT
    orig_start = jnp.cumsum(counts) - counts
    order = jnp.argsort(idx_flat, stable=True)
    sorted_e = idx_flat[order]
    dest_row = padded_start[sorted_e] + (jnp.arange(N, dtype=jnp.int32)
                                         - orig_start[sorted_e])
    src_tok = (jnp.arange(NP, dtype=jnp.int32) % T).at[dest_row].set(
        (order // A).astype(jnp.int32))
    inv_row = jnp.zeros((N,), jnp.int32).at[order].set(dest_row)
    tile_starts = jnp.concatenate(
        [jnp.zeros((1,), jnp.int32), cum_tiles.astype(jnp.int32)])

    x_s = _make_row_gather(T, D, NP)(x, src_tok)

    grid_spec = pltpu.PrefetchScalarGridSpec(
        num_scalar_prefetch=1,
        grid=(E,),
        in_specs=[
            pl.BlockSpec(memory_space=pl.ANY),
            pl.BlockSpec((1, I, D), lambda e, ts: (e, 0, 0)),
            pl.BlockSpec((1, I, D), lambda e, ts: (e, 1, 0)),
            pl.BlockSpec((1, D, I), lambda e, ts: (e, 0, 0)),
        ],
        out_specs=pl.BlockSpec(memory_space=pl.ANY),
        scratch_shapes=[
            pltpu.VMEM((BT, D), jnp.float32),
            pltpu.VMEM((BT, D), jnp.float32),
            pltpu.VMEM((BT, D), jnp.float32),
            pltpu.SemaphoreType.DMA,
            pltpu.SemaphoreType.DMA,
            pltpu.SemaphoreType.DMA,
        ],
    )
    y_s = pl.pallas_call(
        _ffn_body,
        grid_spec=grid_spec,
        out_shape=jax.ShapeDtypeStruct((NP, D), jnp.float32),
        compiler_params=pltpu.CompilerParams(
            dimension_semantics=("arbitrary",)),
    )(tile_starts, x_s, w13, w13, w2)

    out_flat = _make_row_gather(NP, D, N)(y_s, inv_row)
    return out_flat.reshape(T, A, D)
